# Initial kernel scaffold; baseline (speedup 1.0000x reference)
import jax
import jax.numpy as jnp
from jax.experimental import pallas as pl


def kernel(x, edge_index, weight, W1, b1, gn_gamma, gn_beta, gn_alpha, W2, att_src, att_dst, b2):
    raise NotImplementedError("write your pallas kernel here")


# speedup vs baseline: 15.3076x; 15.3076x over previous
import functools

import jax
import jax.numpy as jnp
from jax import lax
from jax.experimental import pallas as pl
from jax.experimental.pallas import tpu as pltpu
from jax.experimental.pallas import tpu_sc as plsc

NN = 10000
EE = 320000
DD = 128
CH = 128
NCORE = 2
NSUB = 16
NWORK = NCORE * NSUB
NCHUNK = 2560
EPAD = NCHUNK * CH - EE
WCH = NCHUNK // NWORK
BLK = 16
NFULL = WCH // BLK
NPAD = 10240
RPW = NPAD // NSUB
RPW_C = 128

_mesh = functools.partial(
    plsc.VectorSubcoreMesh, core_axis_name="c", subcore_axis_name="s")

_SC_PARAMS = pltpu.CompilerParams(needs_layout_passes=False)


def _worker_ids():
    c = lax.axis_index("c")
    s = lax.axis_index("s")
    wid = c * NSUB + s
    start = WCH * wid
    return c, s, wid, start


def _zero_rows(ref, n, width):
    def body(i, _):
        for j in range(width // 16):
            ref[i, pl.ds(j * 16, 16)] = jnp.zeros((16,), jnp.float32)
        return 0
    lax.fori_loop(0, n, body, 0)


def _zero_flat(ref, n):
    def body(i, _):
        ref[pl.ds(i * 16, 16)] = jnp.zeros((16,), jnp.float32)
        return 0
    lax.fori_loop(0, n // 16, body, 0)


def _load_block(hbm2d, buf, row0, nrows):
    pltpu.sync_copy(hbm2d.at[pl.ds(row0, nrows)], buf.at[pl.ds(0, nrows)])


def _sc_deg_body(dst2_hbm, w2_hbm, out_hbm, dstB, wB, deg_loc):
    c, s, wid, start = _worker_ids()
    _zero_flat(deg_loc, NPAD)

    def run_chunks(nk):
        def chunk(k, _):
            def grp(g, _):
                idx = dstB[k, pl.ds(g * 16, 16)]
                vals = wB[k, pl.ds(g * 16, 16)]
                plsc.addupdate_scatter(deg_loc, [idx], vals)
                return 0
            lax.fori_loop(0, CH // 16, grp, 0)
            return 0
        lax.fori_loop(0, nk, chunk, 0)

    for b in range(NFULL):
        _load_block(dst2_hbm, dstB, start + b * BLK, BLK)
        _load_block(w2_hbm, wB, start + b * BLK, BLK)
        run_chunks(BLK)

    pltpu.sync_copy(deg_loc, out_hbm.at[wid])


_sc_deg = pl.kernel(
    _sc_deg_body,
    compiler_params=_SC_PARAMS,
    out_type=jax.ShapeDtypeStruct((NWORK, NPAD), jnp.float32),
    mesh=_mesh(),
    scratch_types=[
        pltpu.VMEM((BLK, CH), jnp.int32),
        pltpu.VMEM((BLK, CH), jnp.float32),
        pltpu.VMEM((NPAD,), jnp.float32),
    ],
)


def _zero_accum(accS, rows, s):
    z = rows.at[0]
    _zero_rows(z, RPW_C, DD)
    def zc(t, _):
        pltpu.sync_copy(z, accS.at[pl.ds(s * RPW + t * RPW_C, RPW_C)])
        return 0
    lax.fori_loop(0, RPW // RPW_C, zc, 0)


def _scale_chunk(rowp, sc1):
    @plsc.parallel_loop(0, CH, unroll=4)
    def _(i):
        sc = plsc.load_gather(sc1, [jnp.full((16,), i, jnp.int32)])
        for j in range(DD // 16):
            rowp[i, pl.ds(j * 16, 16)] = rowp[i, pl.ds(j * 16, 16)] * sc


def _fill_scal(sc1, scB, k):
    for j in range(CH // 16):
        sc1[pl.ds(j * 16, 16)] = scB[k, pl.ds(j * 16, 16)]


def _row_block(tab_hbm, accS, srcB, dstB, scB, sc1, rows, semg, semsc, nk):
    pltpu.async_copy(tab_hbm.at[srcB.at[0]], rows.at[0], semg.at[0])

    def chunk(k, _):
        p = k & 1
        pltpu.make_async_copy(tab_hbm.at[srcB.at[k]], rows.at[p],
                              semg.at[p]).wait()
        @pl.when(k >= 1)
        def _():
            pltpu.make_async_copy(rows.at[1 - p], accS.at[dstB.at[k - 1]],
                                  semsc.at[1 - p]).wait()
        @pl.when(k + 1 < nk)
        def _():
            pltpu.async_copy(tab_hbm.at[srcB.at[k + 1]], rows.at[1 - p],
                             semg.at[1 - p])
        _fill_scal(sc1, scB, k)
        _scale_chunk(rows.at[p], sc1)
        pltpu.async_copy(rows.at[p], accS.at[dstB.at[k]], semsc.at[p],
                         add=True)
        return 0
    lax.fori_loop(0, nk, chunk, 0)

    q = (nk - 1) & 1
    pltpu.make_async_copy(rows.at[q], accS.at[dstB.at[nk - 1]],
                          semsc.at[q]).wait()


def _row_pass_body(src2_hbm, dst2_hbm, sc2_hbm, tab_hbm, out_hbm,
                   srcB, dstB, scB, sc1, rows, semg, semsc, accS):
    c, s, wid, start = _worker_ids()
    _zero_accum(accS, rows, s)
    plsc.subcore_barrier()

    for b in range(NFULL):
        _load_block(src2_hbm, srcB, start + b * BLK, BLK)
        _load_block(dst2_hbm, dstB, start + b * BLK, BLK)
        _load_block(sc2_hbm, scB, start + b * BLK, BLK)
        _row_block(tab_hbm, accS, srcB, dstB, scB, sc1, rows,
                   semg, semsc, BLK)

    plsc.subcore_barrier()
    pltpu.sync_copy(accS.at[pl.ds(s * RPW, RPW)],
                    out_hbm.at[c, pl.ds(s * RPW, RPW)])


_sc_rows = pl.kernel(
    _row_pass_body,
    compiler_params=_SC_PARAMS,
    out_type=jax.ShapeDtypeStruct((NCORE, NPAD, DD), jnp.float32),
    mesh=_mesh(),
    scratch_types=[
        pltpu.VMEM((BLK, CH), jnp.int32),
        pltpu.VMEM((BLK, CH), jnp.int32),
        pltpu.VMEM((BLK, CH), jnp.float32),
        pltpu.VMEM((CH,), jnp.float32),
        pltpu.VMEM((2, CH, DD), jnp.float32),
        pltpu.SemaphoreType.DMA((2,)),
        pltpu.SemaphoreType.DMA((2,)),
        pltpu.VMEM_SHARED((NPAD, DD), jnp.float32),
    ],
)


def _sc_gat_scal_body(src2_hbm, dst2_hbm, as_hbm, ad_hbm,
                      ex2_hbm, den_hbm,
                      srcB, dstB, exB, asT, adT, den_loc):
    c, s, wid, start = _worker_ids()
    pltpu.sync_copy(as_hbm, asT)
    pltpu.sync_copy(ad_hbm, adT)
    _zero_flat(den_loc, NPAD)

    def run_chunks(nk):
        def chunk(k, _):
            def grp(g, _):
                si = srcB[k, pl.ds(g * 16, 16)]
                di = dstB[k, pl.ds(g * 16, 16)]
                e = plsc.load_gather(asT, [si]) + plsc.load_gather(adT, [di])
                e = jnp.where(e >= 0, e, 0.2 * e)
                ex = jnp.exp(e)
                exB[k, pl.ds(g * 16, 16)] = ex
                plsc.addupdate_scatter(den_loc, [di], ex)
                return 0
            lax.fori_loop(0, CH // 16, grp, 0)
            return 0
        lax.fori_loop(0, nk, chunk, 0)

    for b in range(NFULL):
        _load_block(src2_hbm, srcB, start + b * BLK, BLK)
        _load_block(dst2_hbm, dstB, start + b * BLK, BLK)
        run_chunks(BLK)
        pltpu.sync_copy(exB, ex2_hbm.at[pl.ds(start + b * BLK, BLK)])

    pltpu.sync_copy(den_loc, den_hbm.at[wid])


_sc_gat_scal = pl.kernel(
    _sc_gat_scal_body,
    compiler_params=_SC_PARAMS,
    out_type=(
        jax.ShapeDtypeStruct((NCHUNK, CH), jnp.float32),
        jax.ShapeDtypeStruct((NWORK, NPAD), jnp.float32),
    ),
    mesh=_mesh(),
    scratch_types=[
        pltpu.VMEM((BLK, CH), jnp.int32),
        pltpu.VMEM((BLK, CH), jnp.int32),
        pltpu.VMEM((BLK, CH), jnp.float32),
        pltpu.VMEM((NPAD,), jnp.float32),
        pltpu.VMEM((NPAD,), jnp.float32),
        pltpu.VMEM((NPAD,), jnp.float32),
    ],
)


def _graph_norm_act(h, gamma, beta, alpha):
    mean = jnp.mean(h, axis=0, keepdims=True)
    out = h - alpha * mean
    var = jnp.mean(out * out, axis=0, keepdims=True)
    out = gamma * out / jnp.sqrt(var + 1e-5) + beta
    return jnp.where(out >= 0, out, 0.01 * out)


def _tc1_body(x_ref, w1_ref, deg_ref, g_ref, dinv_ref):
    deg = jnp.sum(deg_ref[...], axis=0)[:NN] + 1.0
    dinv = jnp.where(deg > 0, lax.rsqrt(deg), 0.0)
    h1 = jnp.dot(x_ref[...], w1_ref[...], preferred_element_type=jnp.float32)
    g_ref[...] = h1 * dinv[:, None]
    dinv_ref[...] = jnp.concatenate(
        [dinv, jnp.zeros((NPAD - NN,), jnp.float32)])


def _tc1(x, W1, deg):
    return pl.pallas_call(
        _tc1_body,
        out_shape=(
            jax.ShapeDtypeStruct((NN, DD), jnp.float32),
            jax.ShapeDtypeStruct((NPAD,), jnp.float32),
        ),
    )(x, W1, deg)


def _tc2_body(x_ref, agg_ref, g_ref, dinv_ref, w2_ref, atts_ref, attd_ref,
              gam_ref, bet_ref, alp_ref, b1_ref,
              x1_ref, h2_ref, as_ref, ad_ref):
    dinv = dinv_ref[:NN]
    t = (agg_ref[0, :NN] + agg_ref[1, :NN] + g_ref[...]) * dinv[:, None] \
        + b1_ref[...]
    t = _graph_norm_act(t, gam_ref[...], bet_ref[...], alp_ref[...])
    x1 = x_ref[...] + t
    h2 = jnp.dot(x1, w2_ref[...], preferred_element_type=jnp.float32)
    x1_ref[...] = x1
    h2_ref[...] = h2
    pad = jnp.zeros((NPAD - NN,), jnp.float32)
    a_s = jnp.sum(h2 * atts_ref[...], axis=-1)
    a_d = jnp.sum(h2 * attd_ref[...], axis=-1)
    as_ref[...] = jnp.concatenate([a_s, pad])
    ad_ref[...] = jnp.concatenate([a_d, pad])


def _tc2(x, agg1, g, dinv, W2, atts, attd, gamma, beta, alpha, b1):
    return pl.pallas_call(
        _tc2_body,
        out_shape=(
            jax.ShapeDtypeStruct((NN, DD), jnp.float32),
            jax.ShapeDtypeStruct((NN, DD), jnp.float32),
            jax.ShapeDtypeStruct((NPAD,), jnp.float32),
            jax.ShapeDtypeStruct((NPAD,), jnp.float32),
        ),
    )(x, agg1, g, dinv, W2, atts, attd, gamma, beta, alpha, b1)


def _tc3_body(x1_ref, agg_ref, h2_ref, den_ref, as_ref, ad_ref,
              gam_ref, bet_ref, alp_ref, b2_ref, out_ref):
    a_s = as_ref[:NN]
    a_d = ad_ref[:NN]
    es = a_s + a_d
    es = jnp.where(es >= 0, es, 0.2 * es)
    selfex = jnp.exp(es)
    den = jnp.sum(den_ref[...], axis=0)[:NN] + selfex
    numer = agg_ref[0, :NN] + agg_ref[1, :NN] + h2_ref[...] * selfex[:, None]
    t = numer / (den + 1e-16)[:, None] + b2_ref[...]
    t = _graph_norm_act(t, gam_ref[...], bet_ref[...], alp_ref[...])
    out_ref[...] = x1_ref[...] + t


def _tc3(x1, agg2, h2, den, as1, ad1, gamma, beta, alpha, b2):
    return pl.pallas_call(
        _tc3_body,
        out_shape=jax.ShapeDtypeStruct((NN, DD), jnp.float32),
    )(x1, agg2, h2, den, as1, ad1, gamma, beta, alpha, b2)


def kernel(x, edge_index, weight, W1, b1, gn_gamma, gn_beta, gn_alpha,
           W2, att_src, att_dst, b2):
    src2 = jnp.concatenate(
        [edge_index[0], jnp.zeros((EPAD,), jnp.int32)]).reshape(NCHUNK, CH)
    dst2 = jnp.concatenate(
        [edge_index[1], jnp.full((EPAD,), NN, jnp.int32)]).reshape(NCHUNK, CH)
    w2e = jnp.concatenate(
        [weight, jnp.zeros((EPAD,), jnp.float32)]).reshape(NCHUNK, CH)
    b1r = b1.reshape(1, DD)
    b2r = b2.reshape(1, DD)
    attsr = att_src.reshape(1, DD)
    attdr = att_dst.reshape(1, DD)
    gam = gn_gamma.reshape(1, DD)
    bet = gn_beta.reshape(1, DD)
    alp = gn_alpha.reshape(1, DD)

    deg = _sc_deg(dst2, w2e)
    g, dinv = _tc1(x, W1, deg)
    agg1 = _sc_rows(src2, dst2, w2e, g)
    x1, h2, as1, ad1 = _tc2(x, agg1, g, dinv, W2, attsr, attdr,
                            gam, bet, alp, b1r)
    ex2, den = _sc_gat_scal(src2, dst2, as1, ad1)
    agg2 = _sc_rows(src2, dst2, ex2, h2)
    return _tc3(x1, agg2, h2, den, as1, ad1, gam, bet, alp, b2r)


# speedup vs baseline: 41.6929x; 2.7237x over previous
import functools

import jax
import jax.numpy as jnp
from jax import lax
from jax.experimental import pallas as pl
from jax.experimental.pallas import tpu as pltpu
from jax.experimental.pallas import tpu_sc as plsc

NN = 10000
EE = 320000
DD = 128
CH = 128
NCORE = 2
NSUB = 16
NWORK = NCORE * NSUB
NCHUNK = 2560
EPAD = NCHUNK * CH - EE
WCH = NCHUNK // NWORK
BLK = 16
NFULL = WCH // BLK
NPAD = 10240
RPW = NPAD // NSUB
RPW_C = 128

_mesh = functools.partial(
    plsc.VectorSubcoreMesh, core_axis_name="c", subcore_axis_name="s")

_SC_PARAMS = pltpu.CompilerParams(needs_layout_passes=False)


def _worker_ids():
    c = lax.axis_index("c")
    s = lax.axis_index("s")
    wid = c * NSUB + s
    start = WCH * wid
    return c, s, wid, start


def _zero_rows(ref, n, width):
    def body(i, _):
        for j in range(width // 16):
            ref[i, pl.ds(j * 16, 16)] = jnp.zeros((16,), jnp.float32)
        return 0
    lax.fori_loop(0, n, body, 0)


def _zero_flat(ref, n):
    def body(i, _):
        ref[pl.ds(i * 16, 16)] = jnp.zeros((16,), jnp.float32)
        return 0
    lax.fori_loop(0, n // 16, body, 0)


def _load_block(hbm2d, buf, row0, nrows):
    pltpu.sync_copy(hbm2d.at[pl.ds(row0, nrows)], buf.at[pl.ds(0, nrows)])


def _sc_deg_body(dst2_hbm, w2_hbm, out_hbm, dstB, wB, deg_loc):
    c, s, wid, start = _worker_ids()
    _zero_flat(deg_loc, NPAD)

    def run_chunks(nk):
        def chunk(k, _):
            def grp(g, _):
                idx = dstB[k, pl.ds(g * 16, 16)]
                vals = wB[k, pl.ds(g * 16, 16)]
                plsc.addupdate_scatter(deg_loc, [idx], vals)
                return 0
            lax.fori_loop(0, CH // 16, grp, 0)
            return 0
        lax.fori_loop(0, nk, chunk, 0)

    for b in range(NFULL):
        _load_block(dst2_hbm, dstB, start + b * BLK, BLK)
        _load_block(w2_hbm, wB, start + b * BLK, BLK)
        run_chunks(BLK)

    pltpu.sync_copy(deg_loc, out_hbm.at[wid])


_sc_deg = pl.kernel(
    _sc_deg_body,
    compiler_params=_SC_PARAMS,
    out_type=jax.ShapeDtypeStruct((NWORK, NPAD), jnp.float32),
    mesh=_mesh(),
    scratch_types=[
        pltpu.VMEM((BLK, CH), jnp.int32),
        pltpu.VMEM((BLK, CH), jnp.float32),
        pltpu.VMEM((NPAD,), jnp.float32),
    ],
)


def _zero_accum(accS, rows, s):
    z = rows.at[0]
    _zero_rows(z, RPW_C, DD)
    def zc(t, _):
        pltpu.sync_copy(z, accS.at[pl.ds(s * RPW + t * RPW_C, RPW_C)])
        return 0
    lax.fori_loop(0, RPW // RPW_C, zc, 0)


def _scale_chunk(rowp, sc1):
    @plsc.parallel_loop(0, CH, unroll=4)
    def _(i):
        sc = plsc.load_gather(sc1, [jnp.full((16,), i, jnp.int32)])
        for j in range(DD // 16):
            rowp[i, pl.ds(j * 16, 16)] = rowp[i, pl.ds(j * 16, 16)] * sc


def _fill_scal(sc1, scB, k):
    for j in range(CH // 16):
        sc1[pl.ds(j * 16, 16)] = scB[k, pl.ds(j * 16, 16)]


def _row_block(tab_hbm, accS, srcB, dstB, scB, sc1, rows, semg, semsc, nk):
    pltpu.async_copy(tab_hbm.at[srcB.at[0]], rows.at[0], semg.at[0])

    def chunk(k, _):
        p = k & 1
        pltpu.make_async_copy(tab_hbm.at[srcB.at[k]], rows.at[p],
                              semg.at[p]).wait()
        @pl.when(k >= 1)
        def _():
            pltpu.make_async_copy(rows.at[1 - p], accS.at[dstB.at[k - 1]],
                                  semsc.at[1 - p]).wait()
        @pl.when(k + 1 < nk)
        def _():
            pltpu.async_copy(tab_hbm.at[srcB.at[k + 1]], rows.at[1 - p],
                             semg.at[1 - p])
        _fill_scal(sc1, scB, k)
        _scale_chunk(rows.at[p], sc1)
        pltpu.async_copy(rows.at[p], accS.at[dstB.at[k]], semsc.at[p],
                         add=True)
        return 0
    lax.fori_loop(0, nk, chunk, 0)

    q = (nk - 1) & 1
    pltpu.make_async_copy(rows.at[q], accS.at[dstB.at[nk - 1]],
                          semsc.at[q]).wait()


def _row_pass_body(src2_hbm, dst2_hbm, sc2_hbm, tab_hbm, out_hbm,
                   srcB, dstB, scB, sc1, rows, semg, semsc, accS):
    c, s, wid, start = _worker_ids()
    _zero_accum(accS, rows, s)
    plsc.subcore_barrier()

    for b in range(NFULL):
        _load_block(src2_hbm, srcB, start + b * BLK, BLK)
        _load_block(dst2_hbm, dstB, start + b * BLK, BLK)
        _load_block(sc2_hbm, scB, start + b * BLK, BLK)
        _row_block(tab_hbm, accS, srcB, dstB, scB, sc1, rows,
                   semg, semsc, BLK)

    plsc.subcore_barrier()
    pltpu.sync_copy(accS.at[pl.ds(s * RPW, RPW)],
                    out_hbm.at[c, pl.ds(s * RPW, RPW)])


_sc_rows = pl.kernel(
    _row_pass_body,
    compiler_params=_SC_PARAMS,
    out_type=jax.ShapeDtypeStruct((NCORE, NPAD, DD), jnp.float32),
    mesh=_mesh(),
    scratch_types=[
        pltpu.VMEM((BLK, CH), jnp.int32),
        pltpu.VMEM((BLK, CH), jnp.int32),
        pltpu.VMEM((BLK, CH), jnp.float32),
        pltpu.VMEM((CH,), jnp.float32),
        pltpu.VMEM((2, CH, DD), jnp.float32),
        pltpu.SemaphoreType.DMA((2,)),
        pltpu.SemaphoreType.DMA((2,)),
        pltpu.VMEM_SHARED((NPAD, DD), jnp.float32),
    ],
)


def _sc_gat_scal_body(src2_hbm, dst2_hbm, as_hbm, ad_hbm,
                      ex2_hbm, den_hbm,
                      srcB, dstB, exB, asT, adT, den_loc):
    c, s, wid, start = _worker_ids()
    pltpu.sync_copy(as_hbm, asT)
    pltpu.sync_copy(ad_hbm, adT)
    _zero_flat(den_loc, NPAD)

    def run_chunks(nk):
        def chunk(k, _):
            def grp(g, _):
                si = srcB[k, pl.ds(g * 16, 16)]
                di = dstB[k, pl.ds(g * 16, 16)]
                e = plsc.load_gather(asT, [si]) + plsc.load_gather(adT, [di])
                e = jnp.where(e >= 0, e, 0.2 * e)
                ex = jnp.exp(e)
                exB[k, pl.ds(g * 16, 16)] = ex
                plsc.addupdate_scatter(den_loc, [di], ex)
                return 0
            lax.fori_loop(0, CH // 16, grp, 0)
            return 0
        lax.fori_loop(0, nk, chunk, 0)

    for b in range(NFULL):
        _load_block(src2_hbm, srcB, start + b * BLK, BLK)
        _load_block(dst2_hbm, dstB, start + b * BLK, BLK)
        run_chunks(BLK)
        pltpu.sync_copy(exB, ex2_hbm.at[pl.ds(start + b * BLK, BLK)])

    pltpu.sync_copy(den_loc, den_hbm.at[wid])


_sc_gat_scal = pl.kernel(
    _sc_gat_scal_body,
    compiler_params=_SC_PARAMS,
    out_type=(
        jax.ShapeDtypeStruct((NCHUNK, CH), jnp.float32),
        jax.ShapeDtypeStruct((NWORK, NPAD), jnp.float32),
    ),
    mesh=_mesh(),
    scratch_types=[
        pltpu.VMEM((BLK, CH), jnp.int32),
        pltpu.VMEM((BLK, CH), jnp.int32),
        pltpu.VMEM((BLK, CH), jnp.float32),
        pltpu.VMEM((NPAD,), jnp.float32),
        pltpu.VMEM((NPAD,), jnp.float32),
        pltpu.VMEM((NPAD,), jnp.float32),
    ],
)


def _graph_norm_act(h, gamma, beta, alpha):
    mean = jnp.mean(h, axis=0, keepdims=True)
    out = h - alpha * mean
    var = jnp.mean(out * out, axis=0, keepdims=True)
    out = gamma * out / jnp.sqrt(var + 1e-5) + beta
    return jnp.where(out >= 0, out, 0.01 * out)


def _tc1_body(x_ref, w1_ref, deg_ref, g_ref, dinv_ref):
    deg = jnp.sum(deg_ref[...], axis=0)[:NN] + 1.0
    dinv = jnp.where(deg > 0, lax.rsqrt(deg), 0.0)
    h1 = jnp.dot(x_ref[...], w1_ref[...], preferred_element_type=jnp.float32)
    g_ref[...] = h1 * dinv[:, None]
    dinv_ref[...] = jnp.concatenate(
        [dinv, jnp.zeros((NPAD - NN,), jnp.float32)])


def _tc1(x, W1, deg):
    return pl.pallas_call(
        _tc1_body,
        out_shape=(
            jax.ShapeDtypeStruct((NN, DD), jnp.float32),
            jax.ShapeDtypeStruct((NPAD,), jnp.float32),
        ),
    )(x, W1, deg)


def _tc2_body(x_ref, agg_ref, g_ref, dinv_ref, w2_ref, atts_ref, attd_ref,
              gam_ref, bet_ref, alp_ref, b1_ref,
              x1_ref, h2_ref, as_ref, ad_ref):
    dinv = dinv_ref[:NN]
    t = (agg_ref[0, :NN] + agg_ref[1, :NN] + g_ref[...]) * dinv[:, None] \
        + b1_ref[...]
    t = _graph_norm_act(t, gam_ref[...], bet_ref[...], alp_ref[...])
    x1 = x_ref[...] + t
    h2 = jnp.dot(x1, w2_ref[...], preferred_element_type=jnp.float32)
    x1_ref[...] = x1
    h2_ref[...] = h2
    pad = jnp.zeros((NPAD - NN,), jnp.float32)
    a_s = jnp.sum(h2 * atts_ref[...], axis=-1)
    a_d = jnp.sum(h2 * attd_ref[...], axis=-1)
    as_ref[...] = jnp.concatenate([a_s, pad])
    ad_ref[...] = jnp.concatenate([a_d, pad])


def _tc2(x, agg1, g, dinv, W2, atts, attd, gamma, beta, alpha, b1):
    return pl.pallas_call(
        _tc2_body,
        out_shape=(
            jax.ShapeDtypeStruct((NN, DD), jnp.float32),
            jax.ShapeDtypeStruct((NN, DD), jnp.float32),
            jax.ShapeDtypeStruct((NPAD,), jnp.float32),
            jax.ShapeDtypeStruct((NPAD,), jnp.float32),
        ),
    )(x, agg1, g, dinv, W2, atts, attd, gamma, beta, alpha, b1)


def _tc3_body(x1_ref, agg_ref, h2_ref, den_ref, as_ref, ad_ref,
              gam_ref, bet_ref, alp_ref, b2_ref, out_ref):
    a_s = as_ref[:NN]
    a_d = ad_ref[:NN]
    es = a_s + a_d
    es = jnp.where(es >= 0, es, 0.2 * es)
    selfex = jnp.exp(es)
    den = jnp.sum(den_ref[...], axis=0)[:NN] + selfex
    numer = agg_ref[0, :NN] + agg_ref[1, :NN] + h2_ref[...] * selfex[:, None]
    t = numer / (den + 1e-16)[:, None] + b2_ref[...]
    t = _graph_norm_act(t, gam_ref[...], bet_ref[...], alp_ref[...])
    out_ref[...] = x1_ref[...] + t


def _tc3(x1, agg2, h2, den, as1, ad1, gamma, beta, alpha, b2):
    return pl.pallas_call(
        _tc3_body,
        out_shape=jax.ShapeDtypeStruct((NN, DD), jnp.float32),
    )(x1, agg2, h2, den, as1, ad1, gamma, beta, alpha, b2)


def kernel(x, edge_index, weight, W1, b1, gn_gamma, gn_beta, gn_alpha,
           W2, att_src, att_dst, b2):
    pad_ar = jnp.arange(EPAD, dtype=jnp.int32)
    src2 = jnp.concatenate(
        [edge_index[0], pad_ar % NN]).reshape(NCHUNK, CH)
    dst2 = jnp.concatenate(
        [edge_index[1], NN + pad_ar % (NPAD - NN)]).reshape(NCHUNK, CH)
    w2e = jnp.concatenate(
        [weight, jnp.zeros((EPAD,), jnp.float32)]).reshape(NCHUNK, CH)
    b1r = b1.reshape(1, DD)
    b2r = b2.reshape(1, DD)
    attsr = att_src.reshape(1, DD)
    attdr = att_dst.reshape(1, DD)
    gam = gn_gamma.reshape(1, DD)
    bet = gn_beta.reshape(1, DD)
    alp = gn_alpha.reshape(1, DD)

    deg = _sc_deg(dst2, w2e)
    g, dinv = _tc1(x, W1, deg)
    agg1 = _sc_rows(src2, dst2, w2e, g)
    x1, h2, as1, ad1 = _tc2(x, agg1, g, dinv, W2, attsr, attdr,
                            gam, bet, alp, b1r)
    ex2, den = _sc_gat_scal(src2, dst2, as1, ad1)
    agg2 = _sc_rows(src2, dst2, ex2, h2)
    return _tc3(x1, agg2, h2, den, as1, ad1, gam, bet, alp, b2r)


# speedup vs baseline: 43.2811x; 1.0381x over previous
import functools

import jax
import jax.numpy as jnp
from jax import lax
from jax.experimental import pallas as pl
from jax.experimental.pallas import tpu as pltpu
from jax.experimental.pallas import tpu_sc as plsc

NN = 10000
EE = 320000
DD = 128
CH = 128
NCORE = 2
NSUB = 16
NWORK = NCORE * NSUB
NCHUNK = 2560
EPAD = NCHUNK * CH - EE
WCH = NCHUNK // NWORK
BLK = 16
NFULL = WCH // BLK
NPAD = 10240
RPW = NPAD // NSUB
RPW_C = 128

_mesh = functools.partial(
    plsc.VectorSubcoreMesh, core_axis_name="c", subcore_axis_name="s")

_SC_PARAMS = pltpu.CompilerParams(needs_layout_passes=False)


def _worker_ids():
    c = lax.axis_index("c")
    s = lax.axis_index("s")
    wid = c * NSUB + s
    start = WCH * wid
    return c, s, wid, start


def _zero_rows(ref, n, width):
    def body(i, _):
        for j in range(width // 16):
            ref[i, pl.ds(j * 16, 16)] = jnp.zeros((16,), jnp.float32)
        return 0
    lax.fori_loop(0, n, body, 0)


def _zero_flat(ref, n):
    def body(i, _):
        ref[pl.ds(i * 16, 16)] = jnp.zeros((16,), jnp.float32)
        return 0
    lax.fori_loop(0, n // 16, body, 0)


def _load_block(hbm2d, buf, row0, nrows):
    pltpu.sync_copy(hbm2d.at[pl.ds(row0, nrows)], buf.at[pl.ds(0, nrows)])


def _sc_deg_body(dst2_hbm, w2_hbm, out_hbm, dstB, wB, deg_loc):
    c, s, wid, start = _worker_ids()
    _zero_flat(deg_loc, NPAD)

    def run_chunks(nk):
        def chunk(k, _):
            def grp(g, _):
                idx = dstB[k, pl.ds(g * 16, 16)]
                vals = wB[k, pl.ds(g * 16, 16)]
                plsc.addupdate_scatter(deg_loc, [idx], vals)
                return 0
            lax.fori_loop(0, CH // 16, grp, 0)
            return 0
        lax.fori_loop(0, nk, chunk, 0)

    for b in range(NFULL):
        _load_block(dst2_hbm, dstB, start + b * BLK, BLK)
        _load_block(w2_hbm, wB, start + b * BLK, BLK)
        run_chunks(BLK)

    pltpu.sync_copy(deg_loc, out_hbm.at[wid])


_sc_deg = pl.kernel(
    _sc_deg_body,
    compiler_params=_SC_PARAMS,
    out_type=jax.ShapeDtypeStruct((NWORK, NPAD), jnp.float32),
    mesh=_mesh(),
    scratch_types=[
        pltpu.VMEM((BLK, CH), jnp.int32),
        pltpu.VMEM((BLK, CH), jnp.float32),
        pltpu.VMEM((NPAD,), jnp.float32),
    ],
)


def _zero_accum(accS, rows, s):
    z = rows.at[0]
    _zero_rows(z, RPW_C, DD)
    def zc(t, _):
        pltpu.sync_copy(z, accS.at[pl.ds(s * RPW + t * RPW_C, RPW_C)])
        return 0
    lax.fori_loop(0, RPW // RPW_C, zc, 0)


def _scale_chunk(rowp, sc1):
    @plsc.parallel_loop(0, CH, unroll=4)
    def _(i):
        sc = plsc.load_gather(sc1, [jnp.full((16,), i, jnp.int32)])
        for j in range(DD // 16):
            rowp[i, pl.ds(j * 16, 16)] = rowp[i, pl.ds(j * 16, 16)] * sc


def _fill_scal(sc1, scB, k):
    for j in range(CH // 16):
        sc1[pl.ds(j * 16, 16)] = scB[k, pl.ds(j * 16, 16)]


def _row_block(tab_hbm, accS, srcB, dstB, scB, sc1, rows, semg, semsc,
               bp, nk):
    pltpu.async_copy(tab_hbm.at[srcB.at[bp, 0]], rows.at[0], semg.at[0])

    def chunk(k, _):
        p = k & 1
        pltpu.make_async_copy(tab_hbm.at[srcB.at[bp, k]], rows.at[p],
                              semg.at[p]).wait()
        @pl.when(k >= 1)
        def _():
            pltpu.make_async_copy(rows.at[1 - p], accS.at[dstB.at[bp, k - 1]],
                                  semsc.at[1 - p]).wait()
        @pl.when(k + 1 < nk)
        def _():
            pltpu.async_copy(tab_hbm.at[srcB.at[bp, k + 1]], rows.at[1 - p],
                             semg.at[1 - p])
        _fill_scal(sc1, scB.at[bp], k)
        _scale_chunk(rows.at[p], sc1)
        pltpu.async_copy(rows.at[p], accS.at[dstB.at[bp, k]], semsc.at[p],
                         add=True)
        return 0
    lax.fori_loop(0, nk, chunk, 0)

    q = (nk - 1) & 1
    pltpu.make_async_copy(rows.at[q], accS.at[dstB.at[bp, nk - 1]],
                          semsc.at[q]).wait()


def _row_pass_body(src2_hbm, dst2_hbm, sc2_hbm, tab_hbm, out_hbm,
                   srcB, dstB, scB, sc1, rows, semg, semsc, semi, accS):
    c, s, wid, start = _worker_ids()
    _zero_accum(accS, rows, s)
    plsc.subcore_barrier()

    pltpu.sync_copy(src2_hbm.at[pl.ds(start, BLK)], srcB.at[0])
    pltpu.sync_copy(dst2_hbm.at[pl.ds(start, BLK)], dstB.at[0])
    pltpu.sync_copy(sc2_hbm.at[pl.ds(start, BLK)], scB.at[0])
    for b in range(NFULL):
        bp = b & 1
        if b + 1 < NFULL:
            nxt = start + (b + 1) * BLK
            pltpu.async_copy(src2_hbm.at[pl.ds(nxt, BLK)], srcB.at[1 - bp],
                             semi)
            pltpu.async_copy(dst2_hbm.at[pl.ds(nxt, BLK)], dstB.at[1 - bp],
                             semi)
            pltpu.async_copy(sc2_hbm.at[pl.ds(nxt, BLK)], scB.at[1 - bp],
                             semi)
        _row_block(tab_hbm, accS, srcB, dstB, scB, sc1, rows,
                   semg, semsc, bp, BLK)
        if b + 1 < NFULL:
            nxt = start + (b + 1) * BLK
            pltpu.make_async_copy(src2_hbm.at[pl.ds(nxt, BLK)],
                                  srcB.at[1 - bp], semi).wait()
            pltpu.make_async_copy(dst2_hbm.at[pl.ds(nxt, BLK)],
                                  dstB.at[1 - bp], semi).wait()
            pltpu.make_async_copy(sc2_hbm.at[pl.ds(nxt, BLK)],
                                  scB.at[1 - bp], semi).wait()

    plsc.subcore_barrier()
    pltpu.sync_copy(accS.at[pl.ds(s * RPW, RPW)],
                    out_hbm.at[c, pl.ds(s * RPW, RPW)])


_sc_rows = pl.kernel(
    _row_pass_body,
    compiler_params=_SC_PARAMS,
    out_type=jax.ShapeDtypeStruct((NCORE, NPAD, DD), jnp.float32),
    mesh=_mesh(),
    scratch_types=[
        pltpu.VMEM((2, BLK, CH), jnp.int32),
        pltpu.VMEM((2, BLK, CH), jnp.int32),
        pltpu.VMEM((2, BLK, CH), jnp.float32),
        pltpu.VMEM((CH,), jnp.float32),
        pltpu.VMEM((2, CH, DD), jnp.float32),
        pltpu.SemaphoreType.DMA((2,)),
        pltpu.SemaphoreType.DMA((2,)),
        pltpu.SemaphoreType.DMA,
        pltpu.VMEM_SHARED((NPAD, DD), jnp.float32),
    ],
)


def _sc_gat_scal_body(src2_hbm, dst2_hbm, as_hbm, ad_hbm,
                      ex2_hbm, den_hbm,
                      srcB, dstB, exB, asT, adT, den_loc):
    c, s, wid, start = _worker_ids()
    pltpu.sync_copy(as_hbm, asT)
    pltpu.sync_copy(ad_hbm, adT)
    _zero_flat(den_loc, NPAD)

    def run_chunks(nk):
        def chunk(k, _):
            def grp(g, _):
                si = srcB[k, pl.ds(g * 16, 16)]
                di = dstB[k, pl.ds(g * 16, 16)]
                e = plsc.load_gather(asT, [si]) + plsc.load_gather(adT, [di])
                e = jnp.where(e >= 0, e, 0.2 * e)
                ex = jnp.exp(e)
                exB[k, pl.ds(g * 16, 16)] = ex
                plsc.addupdate_scatter(den_loc, [di], ex)
                return 0
            lax.fori_loop(0, CH // 16, grp, 0)
            return 0
        lax.fori_loop(0, nk, chunk, 0)

    for b in range(NFULL):
        _load_block(src2_hbm, srcB, start + b * BLK, BLK)
        _load_block(dst2_hbm, dstB, start + b * BLK, BLK)
        run_chunks(BLK)
        pltpu.sync_copy(exB, ex2_hbm.at[pl.ds(start + b * BLK, BLK)])

    pltpu.sync_copy(den_loc, den_hbm.at[wid])


_sc_gat_scal = pl.kernel(
    _sc_gat_scal_body,
    compiler_params=_SC_PARAMS,
    out_type=(
        jax.ShapeDtypeStruct((NCHUNK, CH), jnp.float32),
        jax.ShapeDtypeStruct((NWORK, NPAD), jnp.float32),
    ),
    mesh=_mesh(),
    scratch_types=[
        pltpu.VMEM((BLK, CH), jnp.int32),
        pltpu.VMEM((BLK, CH), jnp.int32),
        pltpu.VMEM((BLK, CH), jnp.float32),
        pltpu.VMEM((NPAD,), jnp.float32),
        pltpu.VMEM((NPAD,), jnp.float32),
        pltpu.VMEM((NPAD,), jnp.float32),
    ],
)


def _graph_norm_act(h, gamma, beta, alpha):
    mean = jnp.mean(h, axis=0, keepdims=True)
    out = h - alpha * mean
    var = jnp.mean(out * out, axis=0, keepdims=True)
    out = gamma * out / jnp.sqrt(var + 1e-5) + beta
    return jnp.where(out >= 0, out, 0.01 * out)


def _tc1_body(x_ref, w1_ref, deg_ref, g_ref, dinv_ref):
    deg = jnp.sum(deg_ref[...], axis=0)[:NN] + 1.0
    dinv = jnp.where(deg > 0, lax.rsqrt(deg), 0.0)
    h1 = jnp.dot(x_ref[...], w1_ref[...], preferred_element_type=jnp.float32)
    g_ref[...] = h1 * dinv[:, None]
    dinv_ref[...] = jnp.concatenate(
        [dinv, jnp.zeros((NPAD - NN,), jnp.float32)])


def _tc1(x, W1, deg):
    return pl.pallas_call(
        _tc1_body,
        out_shape=(
            jax.ShapeDtypeStruct((NN, DD), jnp.float32),
            jax.ShapeDtypeStruct((NPAD,), jnp.float32),
        ),
    )(x, W1, deg)


def _tc2_body(x_ref, agg_ref, g_ref, dinv_ref, w2_ref, atts_ref, attd_ref,
              gam_ref, bet_ref, alp_ref, b1_ref,
              x1_ref, h2_ref, as_ref, ad_ref):
    dinv = dinv_ref[:NN]
    t = (agg_ref[0, :NN] + agg_ref[1, :NN] + g_ref[...]) * dinv[:, None] \
        + b1_ref[...]
    t = _graph_norm_act(t, gam_ref[...], bet_ref[...], alp_ref[...])
    x1 = x_ref[...] + t
    h2 = jnp.dot(x1, w2_ref[...], preferred_element_type=jnp.float32)
    x1_ref[...] = x1
    h2_ref[...] = h2
    pad = jnp.zeros((NPAD - NN,), jnp.float32)
    a_s = jnp.sum(h2 * atts_ref[...], axis=-1)
    a_d = jnp.sum(h2 * attd_ref[...], axis=-1)
    as_ref[...] = jnp.concatenate([a_s, pad])
    ad_ref[...] = jnp.concatenate([a_d, pad])


def _tc2(x, agg1, g, dinv, W2, atts, attd, gamma, beta, alpha, b1):
    return pl.pallas_call(
        _tc2_body,
        out_shape=(
            jax.ShapeDtypeStruct((NN, DD), jnp.float32),
            jax.ShapeDtypeStruct((NN, DD), jnp.float32),
            jax.ShapeDtypeStruct((NPAD,), jnp.float32),
            jax.ShapeDtypeStruct((NPAD,), jnp.float32),
        ),
    )(x, agg1, g, dinv, W2, atts, attd, gamma, beta, alpha, b1)


def _tc3_body(x1_ref, agg_ref, h2_ref, den_ref, as_ref, ad_ref,
              gam_ref, bet_ref, alp_ref, b2_ref, out_ref):
    a_s = as_ref[:NN]
    a_d = ad_ref[:NN]
    es = a_s + a_d
    es = jnp.where(es >= 0, es, 0.2 * es)
    selfex = jnp.exp(es)
    den = jnp.sum(den_ref[...], axis=0)[:NN] + selfex
    numer = agg_ref[0, :NN] + agg_ref[1, :NN] + h2_ref[...] * selfex[:, None]
    t = numer / (den + 1e-16)[:, None] + b2_ref[...]
    t = _graph_norm_act(t, gam_ref[...], bet_ref[...], alp_ref[...])
    out_ref[...] = x1_ref[...] + t


def _tc3(x1, agg2, h2, den, as1, ad1, gamma, beta, alpha, b2):
    return pl.pallas_call(
        _tc3_body,
        out_shape=jax.ShapeDtypeStruct((NN, DD), jnp.float32),
    )(x1, agg2, h2, den, as1, ad1, gamma, beta, alpha, b2)


def kernel(x, edge_index, weight, W1, b1, gn_gamma, gn_beta, gn_alpha,
           W2, att_src, att_dst, b2):
    pad_ar = jnp.arange(EPAD, dtype=jnp.int32)
    src2 = jnp.concatenate(
        [edge_index[0], pad_ar % NN]).reshape(NCHUNK, CH)
    dst2 = jnp.concatenate(
        [edge_index[1], NN + pad_ar % (NPAD - NN)]).reshape(NCHUNK, CH)
    w2e = jnp.concatenate(
        [weight, jnp.zeros((EPAD,), jnp.float32)]).reshape(NCHUNK, CH)
    b1r = b1.reshape(1, DD)
    b2r = b2.reshape(1, DD)
    attsr = att_src.reshape(1, DD)
    attdr = att_dst.reshape(1, DD)
    gam = gn_gamma.reshape(1, DD)
    bet = gn_beta.reshape(1, DD)
    alp = gn_alpha.reshape(1, DD)

    deg = _sc_deg(dst2, w2e)
    g, dinv = _tc1(x, W1, deg)
    agg1 = _sc_rows(src2, dst2, w2e, g)
    x1, h2, as1, ad1 = _tc2(x, agg1, g, dinv, W2, attsr, attdr,
                            gam, bet, alp, b1r)
    ex2, den = _sc_gat_scal(src2, dst2, as1, ad1)
    agg2 = _sc_rows(src2, dst2, ex2, h2)
    return _tc3(x1, agg2, h2, den, as1, ad1, gam, bet, alp, b2r)
